# baseline (device time: 63265 ns/iter reference)
import jax
import jax.numpy as jnp
from jax import lax
from jax.experimental import pallas as pl
from jax.experimental.pallas import tpu as pltpu

N_DEV = 8
N_STEP = 3
NS = 4


def kernel(A, B):
    m, _ = A.shape
    _, n = B.shape
    chunk = m // N_DEV
    sub = chunk // NS

    def body(a_ref, b_ref, out_ref, c_r, c_l, z_src, z_dst, b16,
             send_r, recv_r, send_l, recv_l, z_sems):
        d = lax.axis_index("i")
        left = lax.rem(d + N_DEV - 1, N_DEV)
        right = lax.rem(d + 1, N_DEV)
        zpart = lax.rem(d + 4, N_DEV)

        b16[...] = b_ref[...].astype(jnp.bfloat16)

        def dot_chunk(c, out_dtype=jnp.bfloat16):
            p = jnp.dot(
                a_ref[pl.ds(c * chunk, chunk), :].astype(jnp.bfloat16),
                b16[...],
                preferred_element_type=jnp.float32,
            )
            return p.astype(out_dtype) if out_dtype != jnp.float32 else p

        z_src[...] = dot_chunk(lax.rem(d + 4, N_DEV))
        pr = dot_chunk(lax.rem(d + 3, N_DEV))
        pl_ = dot_chunk(lax.rem(d + N_DEV - 3, N_DEV))
        for k in range(NS):
            c_r[k, N_STEP] = pr[k * sub:(k + 1) * sub]
            c_l[k, N_STEP] = pl_[k * sub:(k + 1) * sub]

        barrier_sem = pltpu.get_barrier_semaphore()
        for nbr in (left, right, zpart):
            pl.semaphore_signal(
                barrier_sem, inc=1,
                device_id=(nbr,), device_id_type=pl.DeviceIdType.MESH,
            )
        pl.semaphore_wait(barrier_sem, 3)

        z_rd = pltpu.make_async_remote_copy(
            src_ref=z_src, dst_ref=z_dst,
            send_sem=z_sems.at[0], recv_sem=z_sems.at[1],
            device_id=(zpart,), device_id_type=pl.DeviceIdType.MESH,
        )
        z_rd.start()

        rdmas = {}

        def start_step(q, s):
            k = q // 2
            comm, ssem, nbr = (
                (c_r, send_r, right) if q % 2 == 0 else (c_l, send_l, left)
            )
            rsem = recv_r if q % 2 == 0 else recv_l
            src = N_STEP if s == 0 else s - 1
            rd = pltpu.make_async_remote_copy(
                src_ref=comm.at[k, src], dst_ref=comm.at[k, s],
                send_sem=ssem.at[k, s], recv_sem=rsem.at[k, s],
                device_id=(nbr,), device_id_type=pl.DeviceIdType.MESH,
            )
            rdmas[(q, s)] = rd
            rd.start()

        for q in range(2 * NS):
            start_step(q, 0)

        for s in range(N_STEP - 1):
            pr = dot_chunk(lax.rem(d + 2 - s + N_DEV, N_DEV))
            pl_ = dot_chunk(lax.rem(d - 2 + s + N_DEV, N_DEV))
            for q in range(2 * NS):
                k = q // 2
                comm = c_r if q % 2 == 0 else c_l
                part = (pr if q % 2 == 0 else pl_)[k * sub:(k + 1) * sub]
                rdmas[(q, s)].wait_recv()
                comm[k, s] = comm[k, s] + part
                start_step(q, s + 1)

        pdd = dot_chunk(d, jnp.float32)
        z_rd.wait()
        for q in range(2 * NS):
            if q % 2 != 0:
                continue
            k = q // 2
            rdmas[(q, N_STEP - 1)].wait_recv()
            rdmas[(q + 1, N_STEP - 1)].wait_recv()
            rows = pl.ds(k * sub, sub)
            out_ref[rows, :] = (
                c_r[k, N_STEP - 1].astype(jnp.float32)
                + c_l[k, N_STEP - 1].astype(jnp.float32)
                + z_dst[rows, :].astype(jnp.float32)
                + pdd[k * sub:(k + 1) * sub]
            )

        for q in range(2 * NS):
            for s in range(N_STEP):
                rdmas[(q, s)].wait_send()

    comm_shape = pltpu.VMEM((NS, N_STEP + 1, sub, n), jnp.bfloat16)
    dir_sems = pltpu.SemaphoreType.DMA((NS, N_STEP))
    return pl.pallas_call(
        body,
        out_shape=jax.ShapeDtypeStruct((chunk, n), jnp.float32),
        in_specs=[
            pl.BlockSpec(memory_space=pltpu.VMEM),
            pl.BlockSpec(memory_space=pltpu.VMEM),
        ],
        out_specs=pl.BlockSpec(memory_space=pltpu.VMEM),
        scratch_shapes=[
            comm_shape, comm_shape,
            pltpu.VMEM((chunk, n), jnp.bfloat16),
            pltpu.VMEM((chunk, n), jnp.bfloat16),
            pltpu.VMEM((B.shape[0], n), jnp.bfloat16),
            dir_sems, dir_sems,
            dir_sems, dir_sems,
            pltpu.SemaphoreType.DMA((2,)),
        ],
        compiler_params=pltpu.CompilerParams(collective_id=0),
    )(A, B)


# device time: 55460 ns/iter; 1.1407x vs baseline; 1.1407x over previous
import jax
import jax.numpy as jnp
from jax import lax
from jax.experimental import pallas as pl
from jax.experimental.pallas import tpu as pltpu

N_DEV = 8
N_HOP = N_DEV - 1
NS = 4


def kernel(A, B):
    m, _ = A.shape
    _, n = B.shape
    chunk = m // N_DEV
    half = chunk // 2
    sub = half // NS

    def body(a_ref, b_ref, out_ref, c_r, c_l, b16,
             send_r, recv_r, send_l, recv_l):
        d = lax.axis_index("i")

        def sig(x):
            return jnp.where(x < 4, x, 11 - x)

        r = sig(d)
        left = sig(lax.rem(r + N_DEV - 1, N_DEV))
        right = sig(lax.rem(r + 1, N_DEV))

        b16[...] = b_ref[...].astype(jnp.bfloat16)

        def dot_rows(row0, out_dtype=jnp.bfloat16):
            p = jnp.dot(
                a_ref[pl.ds(row0, half), :].astype(jnp.bfloat16), b16[...],
                preferred_element_type=jnp.float32,
            )
            return p.astype(out_dtype) if out_dtype != jnp.float32 else p

        def stream(q):
            k = q // 2
            if q % 2 == 0:
                return c_r, send_r, recv_r, right, k * sub, k
            return c_l, send_l, recv_l, left, half + k * sub, k

        rdmas = {}

        def start_hop(q, s):
            comm, ssem, rsem, nbr, _, k = stream(q)
            src = N_HOP if s == 0 else s - 1
            rd = pltpu.make_async_remote_copy(
                src_ref=comm.at[k, src], dst_ref=comm.at[k, s],
                send_sem=ssem.at[k, s], recv_sem=rsem.at[k, s],
                device_id=(nbr,), device_id_type=pl.DeviceIdType.MESH,
            )
            rdmas[(q, s)] = rd
            rd.start()

        pr = dot_rows(sig(lax.rem(r + N_DEV - 1, N_DEV)) * chunk)
        pl_ = dot_rows(sig(lax.rem(r + 1, N_DEV)) * chunk + half)
        for k in range(NS):
            c_r[k, N_HOP] = pr[k * sub:(k + 1) * sub]
            c_l[k, N_HOP] = pl_[k * sub:(k + 1) * sub]

        barrier_sem = pltpu.get_barrier_semaphore()
        for nbr in (left, right):
            pl.semaphore_signal(
                barrier_sem, inc=1,
                device_id=(nbr,), device_id_type=pl.DeviceIdType.MESH,
            )
        pl.semaphore_wait(barrier_sem, 2)

        for q in range(2 * NS):
            start_hop(q, 0)

        for s in range(N_HOP):
            cr = sig(lax.rem(r + 2 * N_DEV - s - 2, N_DEV))
            cl = sig(lax.rem(r + s + 2, N_DEV))
            last = s == N_HOP - 1
            dt = jnp.float32 if last else jnp.bfloat16
            pr = dot_rows(cr * chunk, dt)
            pl_ = dot_rows(cl * chunk + half, dt)
            for q in range(2 * NS):
                comm, _, _, _, out_row, k = stream(q)
                part = (pr if q % 2 == 0 else pl_)[k * sub:(k + 1) * sub]
                rdmas[(q, s)].wait_recv()
                if not last:
                    comm[k, s] = comm[k, s] + part
                    start_hop(q, s + 1)
                else:
                    out_ref[pl.ds(out_row, sub), :] = (
                        comm[k, s].astype(jnp.float32) + part
                    )

        for q in range(2 * NS):
            for s in range(N_HOP):
                rdmas[(q, s)].wait_send()

    comm_shape = pltpu.VMEM((NS, N_DEV, sub, n), jnp.bfloat16)
    dir_sems = pltpu.SemaphoreType.DMA((NS, N_HOP))
    return pl.pallas_call(
        body,
        out_shape=jax.ShapeDtypeStruct((chunk, n), jnp.float32),
        in_specs=[
            pl.BlockSpec(memory_space=pltpu.VMEM),
            pl.BlockSpec(memory_space=pltpu.VMEM),
        ],
        out_specs=pl.BlockSpec(memory_space=pltpu.VMEM),
        scratch_shapes=[
            comm_shape, comm_shape,
            pltpu.VMEM((B.shape[0], n), jnp.bfloat16),
            dir_sems, dir_sems,
            dir_sems, dir_sems,
        ],
        compiler_params=pltpu.CompilerParams(collective_id=0),
    )(A, B)


# device time: 50885 ns/iter; 1.2433x vs baseline; 1.0899x over previous
import jax
import jax.numpy as jnp
from jax import lax
from jax.experimental import pallas as pl
from jax.experimental.pallas import tpu as pltpu

N_DEV = 8
MX, MY, MZ = 1, 3, 4

PARTS = (
    (0, 96, (MX, MY, MZ)),
    (96, 80, (MY, MZ, MX)),
    (176, 80, (MZ, MX, MY)),
)


def kernel(A, B):
    m, _ = A.shape
    _, n = B.shape
    chunk = m // N_DEV

    def body(a_ref, b_ref, out_ref,
             snd0, rcv0, snd1, rcv1, snd2, rcv2, b16,
             ss0, rs0, ss1, rs1, ss2, rs2):
        d = lax.axis_index("i")
        snd = (snd0, snd1, snd2)
        rcv = (rcv0, rcv1, rcv2)
        ssem = (ss0, ss1, ss2)
        rsem = (rs0, rs1, rs2)

        b16[...] = b_ref[...].astype(jnp.bfloat16)

        def dot_part(mask, row0, nrows, out_dtype=jnp.bfloat16):
            c = jnp.bitwise_xor(d, mask)
            p = jnp.dot(
                a_ref[pl.ds(c * chunk + row0, nrows), :].astype(jnp.bfloat16),
                b16[...],
                preferred_element_type=jnp.float32,
            )
            return p.astype(out_dtype) if out_dtype != jnp.float32 else p

        barrier_sem = pltpu.get_barrier_semaphore()
        for mask in (MX, MY, MZ):
            pl.semaphore_signal(
                barrier_sem, inc=1,
                device_id=(jnp.bitwise_xor(d, mask),),
                device_id_type=pl.DeviceIdType.MESH,
            )
        pl.semaphore_wait(barrier_sem, 3)

        rdmas = {}

        def exchange(p, mask, src_ref, dst_ref, sem_idx):
            rd = pltpu.make_async_remote_copy(
                src_ref=src_ref, dst_ref=dst_ref,
                send_sem=ssem[p].at[sem_idx], recv_sem=rsem[p].at[sem_idx],
                device_id=(jnp.bitwise_xor(d, mask),),
                device_id_type=pl.DeviceIdType.MESH,
            )
            rdmas[(p, sem_idx)] = rd
            rd.start()

        def gs(masks):
            _, M2, M3 = masks
            return (0, M2, M3, M2 ^ M3)

        for j in range(4):
            for p, (row0, nrows, masks) in enumerate(PARTS):
                M1 = masks[0]
                g = gs(masks)[j]
                snd[p][j] = dot_part(M1 ^ g, row0, nrows)
                exchange(p, M1, snd[p].at[j], rcv[p].at[j], j)

        for j in range(4):
            for p, (row0, nrows, masks) in enumerate(PARTS):
                v = dot_part(gs(masks)[j], row0, nrows)
                rdmas[(p, j)].wait_recv()
                rcv[p][j] = rcv[p][j] + v

        for p, (_, _, masks) in enumerate(PARTS):
            M2 = masks[1]
            exchange(p, M2, rcv[p].at[1], rcv[p].at[4], 4)
            exchange(p, M2, rcv[p].at[3], rcv[p].at[5], 5)
        for p in range(3):
            rdmas[(p, 4)].wait_recv()
            rcv[p][0] = rcv[p][0] + rcv[p][4]
            rdmas[(p, 5)].wait_recv()
            rcv[p][2] = rcv[p][2] + rcv[p][5]

        for p, (_, _, masks) in enumerate(PARTS):
            exchange(p, masks[2], rcv[p].at[2], rcv[p].at[6], 6)
        for p, (row0, nrows, _) in enumerate(PARTS):
            rdmas[(p, 6)].wait_recv()
            out_ref[pl.ds(row0, nrows), :] = (
                rcv[p][0].astype(jnp.float32) + rcv[p][6].astype(jnp.float32)
            )

        for p in range(3):
            for i in range(7):
                rdmas[(p, i)].wait_send()

    scratch = []
    for row0, nrows, masks in PARTS:
        scratch.append(pltpu.VMEM((4, nrows, n), jnp.bfloat16))
        scratch.append(pltpu.VMEM((7, nrows, n), jnp.bfloat16))
    scratch.append(pltpu.VMEM((B.shape[0], n), jnp.bfloat16))
    for _ in range(3):
        scratch.append(pltpu.SemaphoreType.DMA((7,)))
        scratch.append(pltpu.SemaphoreType.DMA((7,)))

    return pl.pallas_call(
        body,
        out_shape=jax.ShapeDtypeStruct((chunk, n), jnp.float32),
        in_specs=[
            pl.BlockSpec(memory_space=pltpu.VMEM),
            pl.BlockSpec(memory_space=pltpu.VMEM),
        ],
        out_specs=pl.BlockSpec(memory_space=pltpu.VMEM),
        scratch_shapes=scratch,
        compiler_params=pltpu.CompilerParams(collective_id=0),
    )(A, B)


# device time: 42820 ns/iter; 1.4775x vs baseline; 1.1883x over previous
import jax
import jax.numpy as jnp
from jax import lax
from jax.experimental import pallas as pl
from jax.experimental.pallas import tpu as pltpu

N_DEV = 8
MX, MY, MZ = 1, 3, 4

PARTS = (
    (0, 96, (MX, MY, MZ)),
    (96, 80, (MY, MZ, MX)),
    (176, 80, (MZ, MX, MY)),
)


def kernel(A, B):
    m, _ = A.shape
    _, n = B.shape
    chunk = m // N_DEV

    def body(a_ref, b_ref, out_ref,
             snd0, rcv0, snd1, rcv1, snd2, rcv2, b16,
             ss0, rs0, ss1, rs1, ss2, rs2):
        d = lax.axis_index("i")
        snd = (snd0, snd1, snd2)
        rcv = (rcv0, rcv1, rcv2)
        ssem = (ss0, ss1, ss2)
        rsem = (rs0, rs1, rs2)

        b16[...] = b_ref[...].astype(jnp.bfloat16)

        def dot_part(mask, row0, nrows, out_dtype=jnp.bfloat16):
            c = jnp.bitwise_xor(d, mask)
            p = jnp.dot(
                a_ref[pl.ds(c * chunk + row0, nrows), :].astype(jnp.bfloat16),
                b16[...],
                preferred_element_type=jnp.float32,
            )
            return p.astype(out_dtype) if out_dtype != jnp.float32 else p

        barrier_sem = pltpu.get_barrier_semaphore()
        for mask in (MX, MY, MZ):
            pl.semaphore_signal(
                barrier_sem, inc=1,
                device_id=(jnp.bitwise_xor(d, mask),),
                device_id_type=pl.DeviceIdType.MESH,
            )
        pl.semaphore_wait(barrier_sem, 3)

        rdmas = {}

        def exchange(p, mask, src_ref, dst_ref, sem_idx):
            rd = pltpu.make_async_remote_copy(
                src_ref=src_ref, dst_ref=dst_ref,
                send_sem=ssem[p].at[sem_idx], recv_sem=rsem[p].at[sem_idx],
                device_id=(jnp.bitwise_xor(d, mask),),
                device_id_type=pl.DeviceIdType.MESH,
            )
            rdmas[(p, sem_idx)] = rd
            rd.start()

        def gs(masks):
            _, M2, M3 = masks
            return (0, M2, M3, M2 ^ M3)

        for j in (1, 3, 0, 2):
            for p, (row0, nrows, masks) in enumerate(PARTS):
                M1 = masks[0]
                g = gs(masks)[j]
                snd[p][j] = dot_part(M1 ^ g, row0, nrows)
                exchange(p, M1, snd[p].at[j], rcv[p].at[j], j)

        for j in (1, 3):
            for p, (row0, nrows, masks) in enumerate(PARTS):
                v = dot_part(gs(masks)[j], row0, nrows)
                rdmas[(p, j)].wait_recv()
                rcv[p][j] = rcv[p][j] + v
        for p, (_, _, masks) in enumerate(PARTS):
            M2 = masks[1]
            exchange(p, M2, rcv[p].at[3], rcv[p].at[5], 5)
            exchange(p, M2, rcv[p].at[1], rcv[p].at[4], 4)
        for j in (0, 2):
            for p, (row0, nrows, masks) in enumerate(PARTS):
                v = dot_part(gs(masks)[j], row0, nrows)
                rdmas[(p, j)].wait_recv()
                rcv[p][j] = rcv[p][j] + v

        for p, (_, _, masks) in enumerate(PARTS):
            rdmas[(p, 5)].wait_recv()
            rcv[p][2] = rcv[p][2] + rcv[p][5]
            exchange(p, masks[2], rcv[p].at[2], rcv[p].at[6], 6)
        for p in range(3):
            rdmas[(p, 4)].wait_recv()
            rcv[p][0] = rcv[p][0] + rcv[p][4]

        for p, (row0, nrows, _) in enumerate(PARTS):
            rdmas[(p, 6)].wait_recv()
            out_ref[pl.ds(row0, nrows), :] = (
                rcv[p][0].astype(jnp.float32) + rcv[p][6].astype(jnp.float32)
            )

        for p in range(3):
            for i in range(7):
                rdmas[(p, i)].wait_send()

    scratch = []
    for row0, nrows, masks in PARTS:
        scratch.append(pltpu.VMEM((4, nrows, n), jnp.bfloat16))
        scratch.append(pltpu.VMEM((7, nrows, n), jnp.bfloat16))
    scratch.append(pltpu.VMEM((B.shape[0], n), jnp.bfloat16))
    for _ in range(3):
        scratch.append(pltpu.SemaphoreType.DMA((7,)))
        scratch.append(pltpu.SemaphoreType.DMA((7,)))

    return pl.pallas_call(
        body,
        out_shape=jax.ShapeDtypeStruct((chunk, n), jnp.float32),
        in_specs=[
            pl.BlockSpec(memory_space=pltpu.VMEM),
            pl.BlockSpec(memory_space=pltpu.VMEM),
        ],
        out_specs=pl.BlockSpec(memory_space=pltpu.VMEM),
        scratch_shapes=scratch,
        compiler_params=pltpu.CompilerParams(collective_id=0),
    )(A, B)
